# baseline (speedup 1.0000x reference)
import jax
import jax.numpy as jnp
from jax import lax
from jax.experimental import pallas as pl

_IBLK = 8192
_OBLK = 2048
_SUB = _IBLK // _OBLK
_GPB = _IBLK // 128
_GPO = _OBLK // 128


def _tc_body(us_ref, d1s_ref, d2s_ref, v1_ref, v2_ref, o_ref):
    sub = pl.program_id(0) % _SUB
    us = us_ref[:]
    p2s = jnp.minimum(d2s_ref[:], jnp.maximum(us, 0.0))
    p1s = jnp.minimum(d1s_ref[:], jnp.maximum(us - p2s, 0.0))
    iota_g = jax.lax.broadcasted_iota(jnp.int32, (_GPB, 128), 0)
    dims = (((0,), (0,)), ((), ()))
    for k in range(_GPO):
        g = sub * _GPO + k
        vrows = pl.ds(g * 128, 128)
        orows = pl.ds(k * 128, 128)
        oh = (iota_g == g).astype(jnp.float32)
        e2 = jax.lax.dot_general(p2s, oh, dims,
                                 preferred_element_type=jnp.float32)
        e1 = jax.lax.dot_general(p1s, oh, dims,
                                 preferred_element_type=jnp.float32)
        o_ref[orows, :] = v2_ref[vrows, :] * e2 + v1_ref[vrows, :] * e1


def tc_kernel(u, d1, d2, v1, v2):
    B, R = v1.shape
    G = B // 128
    us = u.reshape(G, 128)
    d1s = d1.reshape(G, 128)
    d2s = d2.reshape(G, 128)
    grid = (B // _OBLK,)
    scal_spec = pl.BlockSpec((_GPB, 128), lambda i: (i // _SUB, 0))
    in_spec = pl.BlockSpec((_IBLK, R), lambda i: (i // _SUB, 0))
    out_spec = pl.BlockSpec((_OBLK, R), lambda i: (i, 0))
    return pl.pallas_call(
        _tc_body,
        grid=grid,
        in_specs=[scal_spec, scal_spec, scal_spec, in_spec, in_spec],
        out_specs=out_spec,
        out_shape=jax.ShapeDtypeStruct((B, R), v1.dtype),
    )(us, d1s, d2s, v1, v2)


def kernel(u, d1, d2, v1, v2):
    return tc_kernel(u.reshape(-1), d1.reshape(-1), d2.reshape(-1), v1, v2)


# speedup vs baseline: 1.2194x; 1.2194x over previous
import jax
import jax.numpy as jnp
from jax import lax
from jax.experimental import pallas as pl

_BLK = 4096
_GPB = _BLK // 128
_G = 128


def _tc_body(us_ref, d1s_ref, d2s_ref, v1_ref, v2_ref, o_ref):
    i = pl.program_id(0)
    us = us_ref[:]
    p2s = jnp.minimum(d2s_ref[:], jnp.maximum(us, 0.0))
    p1s = jnp.minimum(d1s_ref[:], jnp.maximum(us - p2s, 0.0))
    iota_g = jax.lax.broadcasted_iota(jnp.int32, (_G, 128), 0)
    dims = (((0,), (0,)), ((), ()))
    for k in range(_GPB):
        g = i * _GPB + k
        rows = pl.ds(k * 128, 128)
        oh = (iota_g == g).astype(jnp.float32)
        e2 = jax.lax.dot_general(p2s, oh, dims,
                                 preferred_element_type=jnp.float32)
        e1 = jax.lax.dot_general(p1s, oh, dims,
                                 preferred_element_type=jnp.float32)
        o_ref[rows, :] = v2_ref[rows, :] * e2 + v1_ref[rows, :] * e1


def tc_kernel(u, d1, d2, v1, v2):
    B, R = v1.shape
    G = B // 128
    us = u.reshape(G, 128)
    d1s = d1.reshape(G, 128)
    d2s = d2.reshape(G, 128)
    grid = (B // _BLK,)
    scal_spec = pl.BlockSpec((G, 128), lambda i: (0, 0))
    vec_spec = pl.BlockSpec((_BLK, R), lambda i: (i, 0))
    return pl.pallas_call(
        _tc_body,
        grid=grid,
        in_specs=[scal_spec, scal_spec, scal_spec, vec_spec, vec_spec],
        out_specs=vec_spec,
        out_shape=jax.ShapeDtypeStruct((B, R), v1.dtype),
    )(us, d1s, d2s, v1, v2)


def kernel(u, d1, d2, v1, v2):
    return tc_kernel(u.reshape(-1), d1.reshape(-1), d2.reshape(-1), v1, v2)


# speedup vs baseline: 1.4425x; 1.1830x over previous
import jax
import jax.numpy as jnp
from jax import lax
from jax.experimental import pallas as pl

_BLK = 8192
_GPB = _BLK // 128


def _tc_body(us_ref, d1s_ref, d2s_ref, v1_ref, v2_ref, o_ref):
    us = us_ref[:]
    p2s = jnp.minimum(d2s_ref[:], jnp.maximum(us, 0.0))
    p1s = jnp.minimum(d1s_ref[:], jnp.maximum(us - p2s, 0.0))
    iota_g = jax.lax.broadcasted_iota(jnp.int32, (_GPB, 128), 0)
    dims = (((0,), (0,)), ((), ()))
    for g in range(_GPB):
        rows = pl.ds(g * 128, 128)
        oh = (iota_g == g).astype(jnp.float32)
        e2 = jax.lax.dot_general(p2s, oh, dims,
                                 preferred_element_type=jnp.float32)
        e1 = jax.lax.dot_general(p1s, oh, dims,
                                 preferred_element_type=jnp.float32)
        o_ref[rows, :] = v2_ref[rows, :] * e2 + v1_ref[rows, :] * e1


def tc_kernel(u, d1, d2, v1, v2):
    B, R = v1.shape
    G = B // 128
    us = u.reshape(G, 128)
    d1s = d1.reshape(G, 128)
    d2s = d2.reshape(G, 128)
    grid = (B // _BLK,)
    scal_spec = pl.BlockSpec((_GPB, 128), lambda i: (i, 0))
    vec_spec = pl.BlockSpec((_BLK, R), lambda i: (i, 0))
    return pl.pallas_call(
        _tc_body,
        grid=grid,
        in_specs=[scal_spec, scal_spec, scal_spec, vec_spec, vec_spec],
        out_specs=vec_spec,
        out_shape=jax.ShapeDtypeStruct((B, R), v1.dtype),
    )(us, d1s, d2s, v1, v2)


def kernel(u, d1, d2, v1, v2):
    return tc_kernel(u.reshape(-1), d1.reshape(-1), d2.reshape(-1), v1, v2)
